# baseline (speedup 1.0000x reference)
import functools

import jax
import jax.numpy as jnp
from jax import lax
from jax.experimental import pallas as pl
from jax.experimental.pallas import tpu as pltpu
from jax.experimental.pallas import tpu_sc as plsc

_NC = 2
_NS = 16
_LN = 16
_NW = _NC * _NS

_N = 10000
_K = 16
_NPAD = 10240
_PER_W = _NPAD // _NW
_CH = 4
_NCHUNK = _PER_W // _CH
_IDXC = _CH * _K
_RPS = _NPAD // _NS


def _make_gr_body(d, act, dcc):
    dc = dcc

    def body(table_hbm, idx_hbm, w_hbm, out_hbm,
             idx_v, w_v, rows0, rows1, ob0, ob1, shared,
             sem0, sem1, osem0, osem1):
        sid = lax.axis_index("s")
        wid = sid * _NC + lax.axis_index("c")
        base = wid * _PER_W

        pltpu.sync_copy(table_hbm.at[pl.ds(sid * _RPS, _RPS)],
                        shared.at[pl.ds(sid * _RPS, _RPS)])

        pltpu.sync_copy(idx_hbm.at[pl.ds(wid * _NCHUNK, _NCHUNK)], idx_v)
        pltpu.sync_copy(w_hbm.at[pl.ds(wid * _NCHUNK, _NCHUNK)], w_v)

        plsc.subcore_barrier()

        bufs = ((rows0, sem0, ob0, osem0), (rows1, sem1, ob1, osem1))

        def start(c, rows, sem):
            pltpu.async_copy(shared.at[idx_v.at[c]], rows, sem)

        def wait(c, rows, sem):
            pltpu.make_async_copy(shared.at[idx_v.at[c]], rows, sem).wait()

        def out_slice(c):
            return out_hbm.at[pl.ds(base + c * _CH, _CH)]

        def compute(c, rows, ob):
            def node_body(i, carry):
                w_vec = w_v[c, pl.ds(i * _K, _K)]
                r0 = i * _K
                accs = [jnp.zeros((_LN,), jnp.float32) for _ in range(dc)]
                for k in range(_K):
                    wk = w_vec[k]
                    for dci in range(dc):
                        accs[dci] = accs[dci] + wk * rows[r0 + k, pl.ds(dci * _LN, _LN)]
                for dci in range(dc):
                    v = accs[dci] * (1.0 / _K)
                    if act:
                        v = jnp.where(v > 0.0, v,
                                      jnp.exp(jnp.minimum(v, 0.0)) - 1.0)
                    ob[i, pl.ds(dci * _LN, _LN)] = v
                return carry

            lax.fori_loop(0, _CH, node_body, 0)

        start(0, rows0, sem0)
        start(1, rows1, sem1)

        def pair_body(c2, carry):
            for p in range(2):
                rows, sem, ob, osem = bufs[p]
                c = c2 * 2 + p
                wait(c, rows, sem)

                @pl.when(c >= 2)
                def _():
                    pltpu.make_async_copy(ob, out_slice(c), osem).wait()

                compute(c, rows, ob)
                pltpu.async_copy(ob, out_slice(c), osem)

                @pl.when(c + 2 < _NCHUNK)
                def _():
                    start(c + 2, rows, sem)
            return carry

        lax.fori_loop(0, _NCHUNK // 2, pair_body, 0)
        pltpu.make_async_copy(ob0, out_slice(_NCHUNK - 2), osem0).wait()
        pltpu.make_async_copy(ob1, out_slice(_NCHUNK - 1), osem1).wait()

    return body


def _gather_reduce(table, idx2d, w2d, act, dcc=None):
    d = table.shape[1]
    if dcc is None:
        dcc = d // _LN
    mesh = plsc.VectorSubcoreMesh(core_axis_name="c", subcore_axis_name="s")
    f = functools.partial(
        pl.kernel,
        mesh=mesh,
        out_type=jax.ShapeDtypeStruct((_NPAD, d), jnp.float32),
        scratch_types=[
            pltpu.VMEM((_NCHUNK, _IDXC), jnp.int32),
            pltpu.VMEM((_NCHUNK, _IDXC), jnp.float32),
            pltpu.VMEM((_IDXC, d), jnp.float32),
            pltpu.VMEM((_IDXC, d), jnp.float32),
            pltpu.VMEM((_CH, d), jnp.float32),
            pltpu.VMEM((_CH, d), jnp.float32),
            pltpu.VMEM_SHARED((_NPAD, d), jnp.float32),
            pltpu.SemaphoreType.DMA,
            pltpu.SemaphoreType.DMA,
            pltpu.SemaphoreType.DMA,
            pltpu.SemaphoreType.DMA,
        ],
    )(_make_gr_body(d, act, dcc))
    return f(table, idx2d, w2d)


def _mm_body(y_ref, w_ref, o_ref, *, act):
    v = jnp.dot(y_ref[...], w_ref[...], preferred_element_type=jnp.float32)
    if act:
        v = jnp.where(v > 0.0, v, jnp.exp(jnp.minimum(v, 0.0)) - 1.0)
    o_ref[...] = v


def _mm(y, w, act=False):
    n, d = y.shape
    dout = w.shape[1]
    blk = 2048
    return pl.pallas_call(
        functools.partial(_mm_body, act=act),
        grid=(n // blk,),
        in_specs=[pl.BlockSpec((blk, d), lambda i: (i, 0)),
                  pl.BlockSpec((d, dout), lambda i: (0, 0))],
        out_specs=pl.BlockSpec((blk, dout), lambda i: (i, 0)),
        out_shape=jax.ShapeDtypeStruct((n, dout), jnp.float32),
    )(y, w)


def _prep(a0, a2):
    idx = jnp.pad(a0[:, 1:].astype(jnp.int32), ((0, _NPAD - _N), (0, 0)))
    w = jnp.pad(a2[:, 1:].astype(jnp.float32), ((0, _NPAD - _N), (0, 0)))
    return (idx.reshape(_NW * _NCHUNK, _IDXC),
            w.reshape(_NW * _NCHUNK, _IDXC))


def kernel(x, adj1_0, adj1_1, adj1_2, adj2_0, adj2_1, adj2_2, W1, W2,
           r1_G1, r1_G2, r1_B1, r1_B2, r1_r,
           r2_G1, r2_G2, r2_B1, r2_B2, r2_r,
           g1_W, g2_W):
    idx1, w1 = _prep(adj1_0, adj1_2)
    idx2, w2 = _prep(adj2_0, adj2_2)
    xp = jnp.pad(x, ((0, _NPAD - _N), (0, 0)))

    y1 = _gather_reduce(xp, idx1, w1, act=False)
    h1 = _mm(y1, W1, act=True)
    w2dup = jnp.concatenate([W2, W2], axis=1)
    g2 = _mm(h1, w2dup)
    h2 = _gather_reduce(g2, idx2, w2, act=False, dcc=4)
    return h2[:_N, :64]


# speedup vs baseline: 1.1636x; 1.1636x over previous
import functools

import jax
import jax.numpy as jnp
from jax import lax
from jax.experimental import pallas as pl
from jax.experimental.pallas import tpu as pltpu
from jax.experimental.pallas import tpu_sc as plsc

_NC = 2
_NS = 16
_LN = 16
_NW = _NC * _NS

_N = 10000
_KP1 = 17
_K = 16
_NPAD = 10240
_PER_W = _NPAD // _NW
_CH = 4
_NCHUNK = _PER_W // _CH
_IDXC = _CH * _K
_RPS = 632
_RPS_LAST = _N - 15 * _RPS
_LASTW = _NW - 1
_LAST_NODES = _N - _LASTW * _PER_W


def _make_gr_body(dcc):
    def body(table_hbm, a0_hbm, a2_hbm, out_hbm,
             raw_i, raw_w, ilist, rows0, rows1, ob0, ob1, shared,
             ssem, sem0, sem1, osem0, osem1):
        sid = lax.axis_index("s")
        wid = sid * _NC + lax.axis_index("c")
        base = wid * _PER_W
        is_tail = wid == _LASTW
        nn = jnp.where(is_tail, _LAST_NODES, _PER_W)
        nch = jnp.where(is_tail, _LAST_NODES // _CH, _NCHUNK)

        @pl.when(sid < _NS - 1)
        def _():
            pltpu.async_copy(table_hbm.at[pl.ds(sid * _RPS, _RPS)],
                             shared.at[pl.ds(sid * _RPS, _RPS)], ssem)

        @pl.when(sid == _NS - 1)
        def _():
            pltpu.async_copy(table_hbm.at[pl.ds(15 * _RPS, _RPS_LAST)],
                             shared.at[pl.ds(15 * _RPS, _RPS_LAST)], ssem)

        @pl.when(jnp.logical_not(is_tail))
        def _():
            pltpu.sync_copy(a0_hbm.at[pl.ds(base * _KP1, _PER_W * _KP1)], raw_i)
            pltpu.sync_copy(a2_hbm.at[pl.ds(base * _KP1, _PER_W * _KP1)], raw_w)

        @pl.when(is_tail)
        def _():
            pltpu.sync_copy(a0_hbm.at[pl.ds(base * _KP1, _LAST_NODES * _KP1)],
                            raw_i.at[pl.ds(0, _LAST_NODES * _KP1)])
            pltpu.sync_copy(a2_hbm.at[pl.ds(base * _KP1, _LAST_NODES * _KP1)],
                            raw_w.at[pl.ds(0, _LAST_NODES * _KP1)])

        def repack(n, carry):
            c = lax.shift_right_logical(n, 2)
            s = lax.bitwise_and(n, 3)
            ilist[c, pl.ds(s * _K, _K)] = raw_i[pl.ds(n * _KP1 + 1, _K)]
            return carry

        lax.fori_loop(0, nn, repack, 0)

        @pl.when(sid < _NS - 1)
        def _():
            pltpu.make_async_copy(table_hbm.at[pl.ds(sid * _RPS, _RPS)],
                                  shared.at[pl.ds(sid * _RPS, _RPS)],
                                  ssem).wait()

        @pl.when(sid == _NS - 1)
        def _():
            pltpu.make_async_copy(table_hbm.at[pl.ds(15 * _RPS, _RPS_LAST)],
                                  shared.at[pl.ds(15 * _RPS, _RPS_LAST)],
                                  ssem).wait()

        plsc.subcore_barrier()

        bufs = ((rows0, sem0, ob0, osem0), (rows1, sem1, ob1, osem1))

        def start(c, rows, sem):
            pltpu.async_copy(shared.at[ilist.at[c]], rows, sem)

        def wait(c, rows, sem):
            pltpu.make_async_copy(shared.at[ilist.at[c]], rows, sem).wait()

        def out_slice(c):
            return out_hbm.at[pl.ds(base + c * _CH, _CH)]

        def compute(c, rows, ob):
            def node_body(i, carry):
                w_vec = raw_w[pl.ds((c * _CH + i) * _KP1 + 1, _K)]
                r0 = i * _K
                accs = [jnp.zeros((_LN,), jnp.float32) for _ in range(dcc)]
                for k in range(_K):
                    wk = w_vec[k]
                    for dci in range(dcc):
                        accs[dci] = accs[dci] + wk * rows[r0 + k, pl.ds(dci * _LN, _LN)]
                for dci in range(dcc):
                    ob[i, pl.ds(dci * _LN, _LN)] = accs[dci] * (1.0 / _K)
                return carry

            lax.fori_loop(0, _CH, node_body, 0)

        start(0, rows0, sem0)
        start(1, rows1, sem1)

        def pair_body(c2, carry):
            for p in range(2):
                rows, sem, ob, osem = bufs[p]
                c = c2 * 2 + p
                wait(c, rows, sem)

                @pl.when(c >= 2)
                def _():
                    pltpu.make_async_copy(ob, out_slice(c), osem).wait()

                compute(c, rows, ob)
                pltpu.async_copy(ob, out_slice(c), osem)

                @pl.when(c + 2 < nch)
                def _():
                    start(c + 2, rows, sem)
            return carry

        lax.fori_loop(0, nch // 2, pair_body, 0)
        pltpu.make_async_copy(ob0, out_slice(nch - 2), osem0).wait()
        pltpu.make_async_copy(ob1, out_slice(nch - 1), osem1).wait()

    return body


def _gather_reduce(table, a0, a2, dcc):
    d = table.shape[1]
    mesh = plsc.VectorSubcoreMesh(core_axis_name="c", subcore_axis_name="s")
    f = functools.partial(
        pl.kernel,
        mesh=mesh,
        out_type=jax.ShapeDtypeStruct((_NPAD, d), jnp.float32),
        scratch_types=[
            pltpu.VMEM((_PER_W * _KP1,), jnp.int32),
            pltpu.VMEM((_PER_W * _KP1,), jnp.float32),
            pltpu.VMEM((_NCHUNK, _IDXC), jnp.int32),
            pltpu.VMEM((_IDXC, d), jnp.float32),
            pltpu.VMEM((_IDXC, d), jnp.float32),
            pltpu.VMEM((_CH, d), jnp.float32),
            pltpu.VMEM((_CH, d), jnp.float32),
            pltpu.VMEM_SHARED((_N, d), jnp.float32),
            pltpu.SemaphoreType.DMA,
            pltpu.SemaphoreType.DMA,
            pltpu.SemaphoreType.DMA,
            pltpu.SemaphoreType.DMA,
            pltpu.SemaphoreType.DMA,
        ],
    )(_make_gr_body(dcc))
    return f(table, a0, a2)


def _mm_body(y_ref, w1_ref, w2_ref, o_ref):
    v = jnp.dot(y_ref[...], w1_ref[...], preferred_element_type=jnp.float32)
    v = jnp.where(v > 0.0, v, jnp.exp(jnp.minimum(v, 0.0)) - 1.0)
    o_ref[...] = jnp.dot(v, w2_ref[...], preferred_element_type=jnp.float32)


def _mm_fused(y, w1, w2d):
    n, d = y.shape
    blk = 2048
    return pl.pallas_call(
        _mm_body,
        grid=(n // blk,),
        in_specs=[pl.BlockSpec((blk, d), lambda i: (i, 0)),
                  pl.BlockSpec((d, d), lambda i: (0, 0)),
                  pl.BlockSpec((d, d), lambda i: (0, 0))],
        out_specs=pl.BlockSpec((blk, d), lambda i: (i, 0)),
        out_shape=jax.ShapeDtypeStruct((n, d), jnp.float32),
    )(y, w1, w2d)


def kernel(x, adj1_0, adj1_1, adj1_2, adj2_0, adj2_1, adj2_2, W1, W2,
           r1_G1, r1_G2, r1_B1, r1_B2, r1_r,
           r2_G1, r2_G2, r2_B1, r2_B2, r2_r,
           g1_W, g2_W):
    a10 = adj1_0.astype(jnp.int32).reshape(-1)
    a12 = adj1_2.reshape(-1)
    a20 = adj2_0.astype(jnp.int32).reshape(-1)
    a22 = adj2_2.reshape(-1)

    y1 = _gather_reduce(x, a10, a12, dcc=8)
    w2dup = jnp.concatenate([W2, W2], axis=1)
    g2 = _mm_fused(y1, W1, w2dup)
    h2 = _gather_reduce(g2, a20, a22, dcc=4)
    return h2[:_N, :64]
